# Initial kernel scaffold; baseline (speedup 1.0000x reference)
import functools

import jax
import jax.numpy as jnp
import numpy as np
from jax import lax
from jax.experimental import pallas as pl
from jax.experimental.pallas import tpu as pltpu
from jax.experimental.pallas import tpu_sc as plsc

NUM_ENTITIES = 1000000
EMB_DIM = 128
HALF = EMB_DIM // 2
B = 16384

_NC = 2
_NS = 16
_NW = _NC * _NS
_BPW = B // _NW


def _sc_gather(head, rel, tail, entity_emb, relation_emb):
    mesh = plsc.VectorSubcoreMesh(core_axis_name="c", subcore_axis_name="s")

    @functools.partial(
        pl.kernel,
        out_type=(
            jax.ShapeDtypeStruct((B, EMB_DIM), jnp.float32),
            jax.ShapeDtypeStruct((B, EMB_DIM), jnp.float32),
            jax.ShapeDtypeStruct((B, HALF), jnp.float32),
        ),
        mesh=mesh,
        scratch_types=[
            pltpu.VMEM((_BPW,), jnp.int32),
            pltpu.VMEM((_BPW, EMB_DIM), jnp.float32),
            pltpu.VMEM((_BPW, HALF), jnp.float32),
            pltpu.SemaphoreType.DMA,
        ],
    )
    def k(ent_hbm, relemb_hbm, head_hbm, rel_hbm, tail_hbm,
          head_out, tail_out, rel_out, idx_v, rows_v, rows_h_v, sem):
        wid = lax.axis_index("s") * _NC + lax.axis_index("c")
        base = wid * _BPW
        sl = pl.ds(base, _BPW)

        pltpu.sync_copy(head_hbm.at[sl], idx_v)
        pltpu.async_copy(ent_hbm.at[idx_v], rows_v, sem).wait()
        pltpu.sync_copy(rows_v, head_out.at[sl])

        pltpu.sync_copy(tail_hbm.at[sl], idx_v)
        pltpu.async_copy(ent_hbm.at[idx_v], rows_v, sem).wait()
        pltpu.sync_copy(rows_v, tail_out.at[sl])

        pltpu.sync_copy(rel_hbm.at[sl], idx_v)
        pltpu.async_copy(relemb_hbm.at[idx_v], rows_h_v, sem).wait()
        pltpu.sync_copy(rows_h_v, rel_out.at[sl])

    return k(entity_emb, relation_emb, head, rel, tail)


def _rotate_norm_kernel(head_ref, tail_ref, rel_ref, gamma_ref, out_ref):
    re_h = head_ref[:, :HALF]
    im_h = head_ref[:, HALF:]
    re_t = tail_ref[:, :HALF]
    im_t = tail_ref[:, HALF:]
    ph = rel_ref[...] * np.float32(1.0 / (2.0 * np.pi))
    re_r = jnp.cos(ph)
    im_r = jnp.sin(ph)
    re_d = re_h * re_r - im_h * im_r - re_t
    im_d = re_h * im_r + im_h * re_r - im_t
    s = jnp.sum(re_d * re_d + im_d * im_d, axis=1, keepdims=True)
    out_ref[...] = gamma_ref[0, 0] - jnp.sqrt(s)


def kernel(head, rel, tail, entity_emb, relation_emb, gamma):
    head_rows, tail_rows, rel_rows = _sc_gather(
        head, rel, tail, entity_emb, relation_emb)

    rows_per_blk = 1024
    grid = (B // rows_per_blk,)
    out = pl.pallas_call(
        _rotate_norm_kernel,
        grid=grid,
        in_specs=[
            pl.BlockSpec((rows_per_blk, EMB_DIM), lambda i: (i, 0)),
            pl.BlockSpec((rows_per_blk, EMB_DIM), lambda i: (i, 0)),
            pl.BlockSpec((rows_per_blk, HALF), lambda i: (i, 0)),
            pl.BlockSpec((1, 1), lambda i: (0, 0)),
        ],
        out_specs=pl.BlockSpec((rows_per_blk, 1), lambda i: (i, 0)),
        out_shape=jax.ShapeDtypeStruct((B, 1), jnp.float32),
    )(head_rows, tail_rows, rel_rows, gamma.reshape(1, 1))
    return out.reshape(B)


# speedup vs baseline: 1.5027x; 1.5027x over previous
import functools

import jax
import jax.numpy as jnp
import numpy as np
from jax import lax
from jax.experimental import pallas as pl
from jax.experimental.pallas import tpu as pltpu
from jax.experimental.pallas import tpu_sc as plsc

NUM_ENTITIES = 1000000
EMB_DIM = 128
HALF = EMB_DIM // 2
B = 16384

_NC = 2
_NS = 16
_NW = _NC * _NS
_BPW = B // _NW


def _sc_gather(head, rel, tail, entity_emb, relation_emb_pad):
    mesh = plsc.VectorSubcoreMesh(core_axis_name="c", subcore_axis_name="s")

    @functools.partial(
        pl.kernel,
        out_type=(
            jax.ShapeDtypeStruct((B, EMB_DIM), jnp.float32),
            jax.ShapeDtypeStruct((B, EMB_DIM), jnp.float32),
            jax.ShapeDtypeStruct((B, EMB_DIM), jnp.float32),
        ),
        mesh=mesh,
        scratch_types=[
            pltpu.VMEM((_BPW,), jnp.int32),
            pltpu.VMEM((_BPW, EMB_DIM), jnp.float32),
            pltpu.SemaphoreType.DMA,
        ],
    )
    def k(ent_hbm, relemb_hbm, head_hbm, rel_hbm, tail_hbm,
          head_out, tail_out, rel_out, idx_v, rows_v, sem):
        wid = lax.axis_index("s") * _NC + lax.axis_index("c")
        base = wid * _BPW
        sl = pl.ds(base, _BPW)

        pltpu.sync_copy(head_hbm.at[sl], idx_v)
        pltpu.async_copy(ent_hbm.at[idx_v], rows_v, sem).wait()
        pltpu.sync_copy(rows_v, head_out.at[sl])

        pltpu.sync_copy(tail_hbm.at[sl], idx_v)
        pltpu.async_copy(ent_hbm.at[idx_v], rows_v, sem).wait()
        pltpu.sync_copy(rows_v, tail_out.at[sl])

        pltpu.sync_copy(rel_hbm.at[sl], idx_v)
        pltpu.async_copy(relemb_hbm.at[idx_v], rows_v, sem).wait()
        pltpu.sync_copy(rows_v, rel_out.at[sl])

    return k(entity_emb, relation_emb_pad, head, rel, tail)


def _rotate_norm_kernel(head_ref, tail_ref, rel_ref, gamma_ref, out_ref):
    re_h = head_ref[:, :HALF]
    im_h = head_ref[:, HALF:]
    re_t = tail_ref[:, :HALF]
    im_t = tail_ref[:, HALF:]
    ph = rel_ref[:, :HALF] * np.float32(1.0 / (2.0 * np.pi))
    re_r = jnp.cos(ph)
    im_r = jnp.sin(ph)
    re_d = re_h * re_r - im_h * im_r - re_t
    im_d = re_h * im_r + im_h * re_r - im_t
    s = jnp.sum(re_d * re_d + im_d * im_d, axis=1, keepdims=True)
    out_ref[...] = gamma_ref[0, 0] - jnp.sqrt(s)


def kernel(head, rel, tail, entity_emb, relation_emb, gamma):
    relation_emb_pad = jnp.pad(relation_emb, ((0, 0), (0, EMB_DIM - HALF)))
    head_rows, tail_rows, rel_rows = _sc_gather(
        head, rel, tail, entity_emb, relation_emb_pad)

    rows_per_blk = 1024
    grid = (B // rows_per_blk,)
    out = pl.pallas_call(
        _rotate_norm_kernel,
        grid=grid,
        in_specs=[
            pl.BlockSpec((rows_per_blk, EMB_DIM), lambda i: (i, 0)),
            pl.BlockSpec((rows_per_blk, EMB_DIM), lambda i: (i, 0)),
            pl.BlockSpec((rows_per_blk, EMB_DIM), lambda i: (i, 0)),
            pl.BlockSpec((1, 1), lambda i: (0, 0)),
        ],
        out_specs=pl.BlockSpec((rows_per_blk, 1), lambda i: (i, 0)),
        out_shape=jax.ShapeDtypeStruct((B, 1), jnp.float32),
    )(head_rows, tail_rows, rel_rows, gamma.reshape(1, 1))
    return out.reshape(B)


# speedup vs baseline: 2.7112x; 1.8042x over previous
import functools

import jax
import jax.numpy as jnp
import numpy as np
from jax import lax
from jax.experimental import pallas as pl
from jax.experimental.pallas import tpu as pltpu
from jax.experimental.pallas import tpu_sc as plsc

NUM_RELATIONS = 1000
EMB_DIM = 128
HALF = EMB_DIM // 2
B = 16384

_NC = 2
_NS = 16
_NW = _NC * _NS
_BPW = B // _NW
_C = 128
_NCHUNK = _BPW // _C


def _trig_kernel(rel_emb_ref, out_ref):
    ph = rel_emb_ref[...] * np.float32(1.0 / (2.0 * np.pi))
    out_ref[:, :HALF] = jnp.cos(ph)
    out_ref[:, HALF:] = jnp.sin(ph)


def _make_trig_table(relation_emb):
    return pl.pallas_call(
        _trig_kernel,
        out_shape=jax.ShapeDtypeStruct((NUM_RELATIONS, EMB_DIM), jnp.float32),
    )(relation_emb)


def _vsqrt(s):
    i = plsc.bitcast(s, jnp.int32)
    r = plsc.bitcast(jnp.int32(0x5F3759DF) - lax.shift_right_logical(i, 1),
                     jnp.float32)
    half_s = s * np.float32(0.5)
    for _ in range(3):
        r = r * (np.float32(1.5) - half_s * r * r)
    return s * r


def _sc_score(head, rel, tail, entity_emb, trig, gamma16):
    mesh = plsc.VectorSubcoreMesh(core_axis_name="c", subcore_axis_name="s")

    @functools.partial(
        pl.kernel,
        out_type=jax.ShapeDtypeStruct((B,), jnp.float32),
        mesh=mesh,
        compiler_params=pltpu.CompilerParams(needs_layout_passes=False),
        scratch_types=[
            pltpu.VMEM((_C,), jnp.int32),
            pltpu.VMEM((_C,), jnp.int32),
            pltpu.VMEM((_C,), jnp.int32),
            pltpu.VMEM((_C, EMB_DIM), jnp.float32),
            pltpu.VMEM((_C, EMB_DIM), jnp.float32),
            pltpu.VMEM((_C, EMB_DIM), jnp.float32),
            pltpu.VMEM((16,), jnp.float32),
            pltpu.VMEM((_C,), jnp.float32),
            pltpu.SemaphoreType.DMA,
        ],
    )
    def k(ent_hbm, trig_hbm, head_hbm, rel_hbm, tail_hbm, gamma_hbm, out_hbm,
          ih_v, it_v, ir_v, hb, tb, rb, gv, sv, sem):
        cid = lax.axis_index("c")
        sid = lax.axis_index("s")
        wid = sid * _NC + cid

        pltpu.sync_copy(gamma_hbm, gv)
        g = gv[...]

        def chunk_body(c, carry):
            base = wid * _BPW + c * _C
            sl = pl.ds(base, _C)
            pltpu.sync_copy(head_hbm.at[sl], ih_v)
            pltpu.sync_copy(tail_hbm.at[sl], it_v)
            pltpu.sync_copy(rel_hbm.at[sl], ir_v)
            cp1 = pltpu.async_copy(ent_hbm.at[ih_v], hb, sem)
            cp2 = pltpu.async_copy(ent_hbm.at[it_v], tb, sem)
            cp3 = pltpu.async_copy(trig_hbm.at[ir_v], rb, sem)
            cp1.wait()
            cp2.wait()
            cp3.wait()

            lane = lax.iota(jnp.int32, 16)
            for grp in range(_C // 16):

                def row_body(rr, sel):
                    r = jnp.int32(grp * 16) + rr
                    acc = jnp.zeros((16,), jnp.float32)
                    for j in range(HALF // 16):
                        lo = pl.ds(j * 16, 16)
                        hi = pl.ds(HALF + j * 16, 16)
                        re_h = hb[r, lo]
                        im_h = hb[r, hi]
                        re_t = tb[r, lo]
                        im_t = tb[r, hi]
                        re_r = rb[r, lo]
                        im_r = rb[r, hi]
                        re_d = re_h * re_r - im_h * im_r - re_t
                        im_d = re_h * im_r + im_h * re_r - im_t
                        acc = acc + re_d * re_d + im_d * im_d
                    tot = jnp.full((16,), jnp.sum(acc), jnp.float32)
                    return jnp.where(lane == rr, tot, sel)

                sel = lax.fori_loop(0, 16, row_body,
                                    jnp.zeros((16,), jnp.float32))
                sv[pl.ds(grp * 16, 16)] = g - _vsqrt(sel)

            pltpu.sync_copy(sv, out_hbm.at[sl])
            return carry

        lax.fori_loop(0, _NCHUNK, chunk_body, jnp.int32(0))

    return k(entity_emb, trig, head, rel, tail, gamma16)


def kernel(head, rel, tail, entity_emb, relation_emb, gamma):
    trig = _make_trig_table(relation_emb)
    gamma16 = jnp.broadcast_to(gamma, (16,))
    return _sc_score(head, rel, tail, entity_emb, trig, gamma16)


# speedup vs baseline: 3.0017x; 1.1072x over previous
import functools

import jax
import jax.numpy as jnp
import numpy as np
from jax import lax
from jax.experimental import pallas as pl
from jax.experimental.pallas import tpu as pltpu
from jax.experimental.pallas import tpu_sc as plsc

NUM_RELATIONS = 1000
EMB_DIM = 128
HALF = EMB_DIM // 2
B = 16384

_NC = 2
_NS = 16
_NW = _NC * _NS
_BPW = B // _NW
_C = 128
_NCHUNK = _BPW // _C


def _trig_kernel(rel_emb_ref, out_ref):
    ph = rel_emb_ref[...] * np.float32(1.0 / (2.0 * np.pi))
    out_ref[:, :HALF] = jnp.cos(ph)
    out_ref[:, HALF:] = jnp.sin(ph)


def _make_trig_table(relation_emb):
    return pl.pallas_call(
        _trig_kernel,
        out_shape=jax.ShapeDtypeStruct((NUM_RELATIONS, EMB_DIM), jnp.float32),
    )(relation_emb)


def _vsqrt(s):
    i = plsc.bitcast(s, jnp.int32)
    r = plsc.bitcast(jnp.int32(0x5F3759DF) - lax.shift_right_logical(i, 1),
                     jnp.float32)
    half_s = s * np.float32(0.5)
    for _ in range(3):
        r = r * (np.float32(1.5) - half_s * r * r)
    return s * r


def _row_sq_dist(hb, tb, rb, r):
    acc = None
    for j in range(HALF // 16):
        lo = pl.ds(j * 16, 16)
        hi = pl.ds(HALF + j * 16, 16)
        re_h = hb[r, lo]
        im_h = hb[r, hi]
        re_t = tb[r, lo]
        im_t = tb[r, hi]
        re_r = rb[r, lo]
        im_r = rb[r, hi]
        re_d = re_h * re_r - im_h * im_r - re_t
        im_d = re_h * im_r + im_h * re_r - im_t
        sq = re_d * re_d + im_d * im_d
        acc = sq if acc is None else acc + sq
    return acc


def _sc_score(head, rel, tail, entity_emb, trig, gamma16):
    mesh = plsc.VectorSubcoreMesh(core_axis_name="c", subcore_axis_name="s")

    @functools.partial(
        pl.kernel,
        out_type=jax.ShapeDtypeStruct((B,), jnp.float32),
        mesh=mesh,
        compiler_params=pltpu.CompilerParams(needs_layout_passes=False),
        scratch_types=[
            pltpu.VMEM((_BPW,), jnp.int32),
            pltpu.VMEM((_BPW,), jnp.int32),
            pltpu.VMEM((_BPW,), jnp.int32),
            pltpu.VMEM((_C, EMB_DIM), jnp.float32),
            pltpu.VMEM((_C, EMB_DIM), jnp.float32),
            pltpu.VMEM((_C, EMB_DIM), jnp.float32),
            pltpu.VMEM((_C, EMB_DIM), jnp.float32),
            pltpu.VMEM((_C, EMB_DIM), jnp.float32),
            pltpu.VMEM((_C, EMB_DIM), jnp.float32),
            pltpu.VMEM((16,), jnp.float32),
            pltpu.VMEM((_C,), jnp.float32),
            pltpu.SemaphoreType.DMA,
            pltpu.SemaphoreType.DMA,
        ],
    )
    def k(ent_hbm, trig_hbm, head_hbm, rel_hbm, tail_hbm, gamma_hbm, out_hbm,
          ihs, its, irs, hb0, tb0, rb0, hb1, tb1, rb1, gv, sv, sem0, sem1):
        cid = lax.axis_index("c")
        sid = lax.axis_index("s")
        wid = sid * _NC + cid
        base = wid * _BPW

        pltpu.sync_copy(head_hbm.at[pl.ds(base, _BPW)], ihs)
        pltpu.sync_copy(tail_hbm.at[pl.ds(base, _BPW)], its)
        pltpu.sync_copy(rel_hbm.at[pl.ds(base, _BPW)], irs)
        pltpu.sync_copy(gamma_hbm, gv)
        g = gv[...]

        bufs = [(hb0, tb0, rb0), (hb1, tb1, rb1)]
        sems = [sem0, sem1]

        lane = lax.iota(jnp.int32, 16)
        bitmasks = [(lane & jnp.int32(1 << b)) != 0 for b in range(4)]

        def issue(c, bufset, sem):
            hb, tb, rb = bufset
            s = pl.ds(c * _C, _C)
            return [
                pltpu.async_copy(ent_hbm.at[ihs.at[s]], hb, sem),
                pltpu.async_copy(ent_hbm.at[its.at[s]], tb, sem),
                pltpu.async_copy(trig_hbm.at[irs.at[s]], rb, sem),
            ]

        cps = issue(0, bufs[0], sems[0])
        for c in range(_NCHUNK):
            nxt = issue(c + 1, bufs[(c + 1) % 2], sems[(c + 1) % 2]) \
                if c + 1 < _NCHUNK else None
            for cp in cps:
                cp.wait()
            hb, tb, rb = bufs[c % 2]

            def group_body(grp, carry):
                partials = {}
                for rr in range(16):
                    acc = _row_sq_dist(hb, tb, rb, grp * 16 + jnp.int32(rr))
                    v = jnp.full((16,), jnp.sum(acc), jnp.float32)
                    lvl = 0
                    while lvl in partials:
                        v = jnp.where(bitmasks[lvl], v, partials.pop(lvl))
                        lvl += 1
                    partials[lvl] = v
                sel = partials[4]
                sv[pl.ds(grp * 16, 16)] = g - _vsqrt(sel)
                return carry

            lax.fori_loop(0, _C // 16, group_body, jnp.int32(0))
            pltpu.sync_copy(sv, out_hbm.at[pl.ds(base + c * _C, _C)])
            cps = nxt

    return k(entity_emb, trig, head, rel, tail, gamma16)


def kernel(head, rel, tail, entity_emb, relation_emb, gamma):
    trig = _make_trig_table(relation_emb)
    gamma16 = jnp.broadcast_to(gamma, (16,))
    return _sc_score(head, rel, tail, entity_emb, trig, gamma16)
